# baseline (speedup 1.0000x reference)
import functools

import jax
import jax.numpy as jnp
from jax import lax
from jax.experimental import pallas as pl
from jax.experimental.pallas import tpu as pltpu
from jax.experimental.pallas import tpu_sc as plsc

B = 16384
V = 1000000
D = 64

NC = 2
NS = 16
NW = NC * NS
BPW = B // NW
FK = 16

BV = 2048
NG = (V + 2 * BV - 1) // (2 * BV)
PR = NG * BV


def _tr_body(x, eye, o):
    lt = jnp.einsum("km,kn->mn", x[:, 0:BV], eye[...],
                    preferred_element_type=jnp.float32)
    rt_ = jnp.einsum("km,kn->mn", x[:, BV:2 * BV], eye[...],
                     preferred_element_type=jnp.float32)
    o[...] = jnp.concatenate([lt, rt_], axis=1)


def _transpose1(t3, eye):
    return pl.pallas_call(
        _tr_body,
        grid=(NG,),
        in_specs=[pl.BlockSpec((D, 2 * BV), lambda g: (0, g)),
                  pl.BlockSpec((D, D), lambda g: (0, 0))],
        out_specs=pl.BlockSpec((BV, 2 * D), lambda g: (g, 0)),
        out_shape=jax.ShapeDtypeStruct((PR, 2 * D), jnp.float32),
    )(t3, eye)


def _gather3_body(rt_hbm, re_hbm, rm_hbm, e1_hbm, e2_hbm, e3_hbm,
                  g1_hbm, g2_hbm, g3_hbm, idx_v, rows_v, rows3_v,
                  sem, sem_out):
    wid = lax.axis_index("s") * NC + lax.axis_index("c")
    base = wid * BPW
    idx_hbms = (rt_hbm, re_hbm, rm_hbm)
    for t in range(3):
        pltpu.sync_copy(idx_hbms[t].at[pl.ds(base, BPW)],
                        idx_v.at[pl.ds(t * BPW, BPW)])

    def do_table(t, tab, out_hbm, rv, half):
        hn = BPW // 2
        off = half * (BPW // 2)

        def fire(g, _):
            idx16 = idx_v[pl.ds(t * BPW + off + g * FK, FK)]
            for u in range(FK):
                j = g * FK + u
                i = idx16[u]
                if t < 2:
                    r = i // 2
                    pltpu.async_copy(tab.at[r // 8, pl.ds(r % 8, 1)],
                                     rv.at[j // 8, pl.ds(j % 8, 1)], sem)
                else:
                    r = (i // 4096) * 2048 + (i % 2048)
                    pltpu.async_copy(tab.at[r // 8, pl.ds(r % 8, 1)],
                                     rv.at[j // 8, pl.ds(j % 8, 1)], sem)
            return ()

        lax.fori_loop(0, hn // FK, fire, (), unroll=False)
        pltpu.make_async_copy(tab.at[pl.ds(0, hn // 8)], rv, sem).wait()
        return pltpu.async_copy(
            rv, out_hbm.at[pl.ds((base + off) // 8, hn // 8)], sem_out)

    prev = None
    for t, tab, out in ((0, e1_hbm, g1_hbm), (1, e2_hbm, g2_hbm),
                        (2, e3_hbm, g3_hbm)):
        for half in (0, 1):
            rv = rows_v if (2 * t + half) % 2 == 0 else rows3_v
            if prev is not None:
                prev.wait()
            prev = do_table(t, tab, out, rv, half)
    prev.wait()


_gather3 = pl.kernel(
    _gather3_body,
    out_type=(jax.ShapeDtypeStruct((B // 8, 8, 2 * D), jnp.float32),) * 3,
    mesh=plsc.VectorSubcoreMesh(core_axis_name="c", subcore_axis_name="s",
                                num_cores=NC, num_subcores=NS),
    scratch_types=[
        pltpu.VMEM((3 * BPW,), jnp.int32),
        pltpu.VMEM((BPW // 16, 8, 2 * D), jnp.float32),
        pltpu.VMEM((BPW // 16, 8, 2 * D), jnp.float32),
        pltpu.SemaphoreType.DMA,
        pltpu.SemaphoreType.DMA,
    ],
)


BM = 2048


def _mm_body(g1, g2, g3, p1, p2, p3, wt, bb, out):
    acc = bb[...]
    for g, p, lo in ((g1, p1, 0), (g2, p2, D), (g3, p3, 2 * D)):
        par = p[...]
        sel = g[:, 0:D] * (1.0 - par) + g[:, D:2 * D] * par
        acc = acc + jnp.dot(sel, wt[lo:lo + D, :],
                            preferred_element_type=jnp.float32)
    out[...] = jnp.maximum(acc, 0.0)


def _mm(g1, g2, g3, p1, p2, p3, wt, bb):
    return pl.pallas_call(
        _mm_body,
        grid=(B // BM,),
        in_specs=[
            pl.BlockSpec((BM, 2 * D), lambda i: (i, 0)),
            pl.BlockSpec((BM, 2 * D), lambda i: (i, 0)),
            pl.BlockSpec((BM, 2 * D), lambda i: (i, 0)),
            pl.BlockSpec((BM, 1), lambda i: (i, 0)),
            pl.BlockSpec((BM, 1), lambda i: (i, 0)),
            pl.BlockSpec((BM, 1), lambda i: (i, 0)),
            pl.BlockSpec((3 * D, D), lambda i: (0, 0)),
            pl.BlockSpec((1, D), lambda i: (0, 0)),
        ],
        out_specs=pl.BlockSpec((BM, D), lambda i: (i, 0)),
        out_shape=jax.ShapeDtypeStruct((B, D), jnp.float32),
    )(g1, g2, g3, p1, p2, p3, wt, bb)


def kernel(rt, re, rm, emb1, emb2, emb3, W, b):
    eye = jnp.eye(D, dtype=jnp.float32)
    P3 = _transpose1(emb3.T, eye)
    g1, g2, g3 = _gather3(rt, re, rm,
                          emb1.reshape(V // 16, 8, 2 * D),
                          emb2.reshape(V // 16, 8, 2 * D),
                          P3.reshape(PR // 8, 8, 2 * D))
    h1 = (rt % 2).astype(jnp.float32).reshape(B, 1)
    h2 = (re % 2).astype(jnp.float32).reshape(B, 1)
    h3 = ((rm // 2048) % 2).astype(jnp.float32).reshape(B, 1)
    wt = W.T
    bb = b.reshape(1, D)
    return _mm(g1.reshape(B, 2 * D), g2.reshape(B, 2 * D),
               g3.reshape(B, 2 * D), h1, h2, h3, wt, bb)


# speedup vs baseline: 2.0401x; 2.0401x over previous
import functools

import jax
import jax.numpy as jnp
from jax import lax
from jax.experimental import pallas as pl
from jax.experimental.pallas import tpu as pltpu
from jax.experimental.pallas import tpu_sc as plsc

B = 16384
V = 1000000
D = 64

NC = 2
NS = 16
NW = NC * NS
BPW = B // NW
FK = 16

BV = 2048
NG = (V + 2 * BV - 1) // (2 * BV)
PR = NG * BV


def _tr_body(x2, x3, eye, o2, o3):
    for x, o in ((x2, o2), (x3, o3)):
        lt = jnp.einsum("km,kn->mn", x[:, 0:BV], eye[...],
                        preferred_element_type=jnp.float32)
        rt_ = jnp.einsum("km,kn->mn", x[:, BV:2 * BV], eye[...],
                         preferred_element_type=jnp.float32)
        o[...] = jnp.concatenate([lt, rt_], axis=1)


def _transpose2(t2, t3, eye):
    spec = pl.BlockSpec((D, 2 * BV), lambda g: (0, g))
    return pl.pallas_call(
        _tr_body,
        grid=(NG,),
        in_specs=[spec, spec, pl.BlockSpec((D, D), lambda g: (0, 0))],
        out_specs=[pl.BlockSpec((BV, 2 * D), lambda g: (g, 0))] * 2,
        out_shape=[jax.ShapeDtypeStruct((PR, 2 * D), jnp.float32)] * 2,
    )(t2, t3, eye)


def _gather3_body(rt_hbm, re_hbm, rm_hbm, e1_hbm, e2_hbm, e3_hbm,
                  g1_hbm, g2_hbm, g3_hbm, idx_v, rows_v, rows3_v,
                  sem, sem_out):
    wid = lax.axis_index("s") * NC + lax.axis_index("c")
    base = wid * BPW
    idx_hbms = (rt_hbm, re_hbm, rm_hbm)
    for t in range(3):
        pltpu.sync_copy(idx_hbms[t].at[pl.ds(base, BPW)],
                        idx_v.at[pl.ds(t * BPW, BPW)])

    def do_table(t, tab, out_hbm, rv, half):
        hn = BPW if t == 0 else BPW // 2
        off = half * (BPW // 2)

        def fire(g, _):
            idx16 = idx_v[pl.ds(t * BPW + off + g * FK, FK)]
            for u in range(FK):
                j = g * FK + u
                i = idx16[u]
                if t < 1:
                    pltpu.async_copy(tab.at[i // 8, pl.ds(i % 8, 1)],
                                     rv.at[j // 8, pl.ds(j % 8, 1)], sem)
                else:
                    r = (i // 4096) * 2048 + (i % 2048)
                    pltpu.async_copy(tab.at[r // 8, pl.ds(r % 8, 1)],
                                     rv.at[j // 8, pl.ds(j % 8, 1)], sem)
            return ()

        lax.fori_loop(0, hn // FK, fire, (), unroll=False)
        pltpu.make_async_copy(tab.at[pl.ds(0, hn // 8)], rv, sem).wait()
        return pltpu.async_copy(
            rv, out_hbm.at[pl.ds((base + off) // 8, hn // 8)], sem_out)

    c1 = do_table(0, e1_hbm, g1_hbm, rows_v, 0)
    prev = do_table(1, e2_hbm, g2_hbm, rows3_v, 0)
    for t, tab, out, half in ((1, e2_hbm, g2_hbm, 1),
                              (2, e3_hbm, g3_hbm, 0),
                              (2, e3_hbm, g3_hbm, 1)):
        prev.wait()
        prev = do_table(t, tab, out, rows3_v, half)
    c1.wait()
    prev.wait()


_gather3 = pl.kernel(
    _gather3_body,
    out_type=(jax.ShapeDtypeStruct((B // 8, 8, D), jnp.float32),
              jax.ShapeDtypeStruct((B // 8, 8, 2 * D), jnp.float32),
              jax.ShapeDtypeStruct((B // 8, 8, 2 * D), jnp.float32)),
    mesh=plsc.VectorSubcoreMesh(core_axis_name="c", subcore_axis_name="s",
                                num_cores=NC, num_subcores=NS),
    scratch_types=[
        pltpu.VMEM((3 * BPW,), jnp.int32),
        pltpu.VMEM((BPW // 8, 8, D), jnp.float32),
        pltpu.VMEM((BPW // 16, 8, 2 * D), jnp.float32),
        pltpu.SemaphoreType.DMA,
        pltpu.SemaphoreType.DMA,
    ],
)


BM = 2048


def _mm_body(g1, g2, g3, p2, p3, wt, bb, out):
    acc = bb[...]
    acc = acc + jnp.dot(g1[...], wt[0:D, :],
                        preferred_element_type=jnp.float32)
    for g, p, lo in ((g2, p2, D), (g3, p3, 2 * D)):
        par = p[...]
        sel = g[:, 0:D] * (1.0 - par) + g[:, D:2 * D] * par
        acc = acc + jnp.dot(sel, wt[lo:lo + D, :],
                            preferred_element_type=jnp.float32)
    out[...] = jnp.maximum(acc, 0.0)


def _mm(g1, g2, g3, p2, p3, wt, bb):
    return pl.pallas_call(
        _mm_body,
        grid=(B // BM,),
        in_specs=[
            pl.BlockSpec((BM, D), lambda i: (i, 0)),
            pl.BlockSpec((BM, 2 * D), lambda i: (i, 0)),
            pl.BlockSpec((BM, 2 * D), lambda i: (i, 0)),
            pl.BlockSpec((BM, 1), lambda i: (i, 0)),
            pl.BlockSpec((BM, 1), lambda i: (i, 0)),
            pl.BlockSpec((3 * D, D), lambda i: (0, 0)),
            pl.BlockSpec((1, D), lambda i: (0, 0)),
        ],
        out_specs=pl.BlockSpec((BM, D), lambda i: (i, 0)),
        out_shape=jax.ShapeDtypeStruct((B, D), jnp.float32),
    )(g1, g2, g3, p2, p3, wt, bb)


def kernel(rt, re, rm, emb1, emb2, emb3, W, b):
    eye = jnp.eye(D, dtype=jnp.float32)
    P2, P3 = _transpose2(emb2.T, emb3.T, eye)
    g1, g2, g3 = _gather3(rt, re, rm,
                          emb1.reshape(V // 8, 8, D),
                          P2.reshape(PR // 8, 8, 2 * D),
                          P3.reshape(PR // 8, 8, 2 * D))
    h2 = ((re // 2048) % 2).astype(jnp.float32).reshape(B, 1)
    h3 = ((rm // 2048) % 2).astype(jnp.float32).reshape(B, 1)
    wt = W.T
    bb = b.reshape(1, D)
    return _mm(g1.reshape(B, D), g2.reshape(B, 2 * D), g3.reshape(B, 2 * D),
               h2, h3, wt, bb)


# speedup vs baseline: 2.2492x; 1.1025x over previous
import functools

import jax
import jax.numpy as jnp
from jax import lax
from jax.experimental import pallas as pl
from jax.experimental.pallas import tpu as pltpu
from jax.experimental.pallas import tpu_sc as plsc

B = 16384
V = 1000000
D = 64

NC = 2
NS = 16
NW = NC * NS
BPW = B // NW
FK = 16

BV = 2048
NG = (V + 2 * BV - 1) // (2 * BV)
PR = NG * BV


def _tr_body(x, eye, o):
    lt = jnp.einsum("km,kn->mn", x[:, 0:BV], eye[...],
                    preferred_element_type=jnp.float32)
    rt_ = jnp.einsum("km,kn->mn", x[:, BV:2 * BV], eye[...],
                     preferred_element_type=jnp.float32)
    o[...] = jnp.concatenate([lt, rt_], axis=1)


def _transpose1(t3, eye):
    return pl.pallas_call(
        _tr_body,
        grid=(NG,),
        in_specs=[pl.BlockSpec((D, 2 * BV), lambda g: (0, g)),
                  pl.BlockSpec((D, D), lambda g: (0, 0))],
        out_specs=pl.BlockSpec((BV, 2 * D), lambda g: (g, 0)),
        out_shape=jax.ShapeDtypeStruct((PR, 2 * D), jnp.float32),
    )(t3, eye)


def _gather3_body(rt_hbm, re_hbm, rm_hbm, e1_hbm, e2_hbm, e3_hbm,
                  g1_hbm, g2_hbm, g3_hbm, idx_v, rows_v, rows3_v,
                  sem, sem_out):
    wid = lax.axis_index("s") * NC + lax.axis_index("c")
    base = wid * BPW
    idx_hbms = (rt_hbm, re_hbm, rm_hbm)
    for t in range(3):
        pltpu.sync_copy(idx_hbms[t].at[pl.ds(base, BPW)],
                        idx_v.at[pl.ds(t * BPW, BPW)])

    def do_table(t, tab, out_hbm, rv, half):
        hn = BPW // 2 if t == 2 else BPW
        off = half * (BPW // 2)

        def fire(g, _):
            idx16 = idx_v[pl.ds(t * BPW + off + g * FK, FK)]
            for u in range(FK):
                j = g * FK + u
                i = idx16[u]
                if t < 2:
                    pltpu.async_copy(tab.at[i // 8, pl.ds(i % 8, 1)],
                                     rv.at[j // 8, pl.ds(j % 8, 1)], sem)
                else:
                    r = (i // 4096) * 2048 + (i % 2048)
                    pltpu.async_copy(tab.at[r // 8, pl.ds(r % 8, 1)],
                                     rv.at[j // 8, pl.ds(j % 8, 1)], sem)
            return ()

        lax.fori_loop(0, hn // FK, fire, (), unroll=False)
        pltpu.make_async_copy(tab.at[pl.ds(0, hn // 8)], rv, sem).wait()
        return pltpu.async_copy(
            rv, out_hbm.at[pl.ds((base + off) // 8, hn // 8)], sem_out)

    c1 = do_table(0, e1_hbm, g1_hbm, rows_v, 0)
    c3a = do_table(2, e3_hbm, g3_hbm, rows3_v, 0)
    c1.wait()
    c2 = do_table(1, e2_hbm, g2_hbm, rows_v, 0)
    c3a.wait()
    c3b = do_table(2, e3_hbm, g3_hbm, rows3_v, 1)
    c2.wait()
    c3b.wait()


_gather3 = pl.kernel(
    _gather3_body,
    out_type=(jax.ShapeDtypeStruct((B // 8, 8, D), jnp.float32),
              jax.ShapeDtypeStruct((B // 8, 8, D), jnp.float32),
              jax.ShapeDtypeStruct((B // 8, 8, 2 * D), jnp.float32)),
    mesh=plsc.VectorSubcoreMesh(core_axis_name="c", subcore_axis_name="s",
                                num_cores=NC, num_subcores=NS),
    scratch_types=[
        pltpu.VMEM((3 * BPW,), jnp.int32),
        pltpu.VMEM((BPW // 8, 8, D), jnp.float32),
        pltpu.VMEM((BPW // 16, 8, 2 * D), jnp.float32),
        pltpu.SemaphoreType.DMA,
        pltpu.SemaphoreType.DMA,
    ],
)


BM = 2048


def _mm_body(g1, g2, g3, p3, wt, bb, out):
    acc = bb[...]
    acc = acc + jnp.dot(g1[...], wt[0:D, :],
                        preferred_element_type=jnp.float32)
    acc = acc + jnp.dot(g2[...], wt[D:2 * D, :],
                        preferred_element_type=jnp.float32)
    par = p3[...]
    sel = g3[:, 0:D] * (1.0 - par) + g3[:, D:2 * D] * par
    acc = acc + jnp.dot(sel, wt[2 * D:3 * D, :],
                        preferred_element_type=jnp.float32)
    out[...] = jnp.maximum(acc, 0.0)


def _mm(g1, g2, g3, p3, wt, bb):
    return pl.pallas_call(
        _mm_body,
        grid=(B // BM,),
        in_specs=[
            pl.BlockSpec((BM, D), lambda i: (i, 0)),
            pl.BlockSpec((BM, D), lambda i: (i, 0)),
            pl.BlockSpec((BM, 2 * D), lambda i: (i, 0)),
            pl.BlockSpec((BM, 1), lambda i: (i, 0)),
            pl.BlockSpec((3 * D, D), lambda i: (0, 0)),
            pl.BlockSpec((1, D), lambda i: (0, 0)),
        ],
        out_specs=pl.BlockSpec((BM, D), lambda i: (i, 0)),
        out_shape=jax.ShapeDtypeStruct((B, D), jnp.float32),
    )(g1, g2, g3, p3, wt, bb)


def kernel(rt, re, rm, emb1, emb2, emb3, W, b):
    eye = jnp.eye(D, dtype=jnp.float32)
    P3 = _transpose1(emb3.T, eye)
    g1, g2, g3 = _gather3(rt, re, rm,
                          emb1.reshape(V // 8, 8, D),
                          emb2.reshape(V // 8, 8, D),
                          P3.reshape(PR // 8, 8, 2 * D))
    h3 = ((rm // 2048) % 2).astype(jnp.float32).reshape(B, 1)
    wt = W.T
    bb = b.reshape(1, D)
    return _mm(g1.reshape(B, D), g2.reshape(B, D), g3.reshape(B, 2 * D),
               h3, wt, bb)
